# Initial kernel scaffold; baseline (speedup 1.0000x reference)
import jax
import jax.numpy as jnp
from jax.experimental import pallas as pl


def kernel(x, edge_index, edge_weight, wx1, wh1, wc1, b1, wx2, wh2, wc2, b2, conv1_w, conv1_b, conv2_w, conv2_b, lin_w, lin_b):
    raise NotImplementedError("write your pallas kernel here")


# speedup vs baseline: 2.8747x; 2.8747x over previous
import functools

import jax
import jax.numpy as jnp
from jax import lax
from jax.experimental import pallas as pl
from jax.experimental.pallas import tpu as pltpu
from jax.experimental.pallas import tpu_sc as plsc

N = 10000
NP = 10112
E = 320000
NTILES = 32
NSUB = 16
EPW = E // NTILES
CHUNK = 80
NPT = NP // NSUB


def _sc_mesh():
    return plsc.VectorSubcoreMesh(core_axis_name="c", subcore_axis_name="s")


def _make_propagate(F, chunk, stage_table):
    scratch = [
        pltpu.VMEM((chunk,), jnp.int32),
        pltpu.VMEM((chunk,), jnp.int32),
        pltpu.VMEM((chunk, 16), jnp.float32),
        pltpu.VMEM((chunk, F), jnp.float32),
        pltpu.VMEM_SHARED((NP, F), jnp.float32),
        pltpu.SemaphoreType.DMA,
    ]
    if stage_table:
        scratch.append(pltpu.VMEM_SHARED((NP, F), jnp.float32))

    @functools.partial(
        pl.kernel,
        out_type=jax.ShapeDtypeStruct((2, NP, F), jnp.float32),
        mesh=_sc_mesh(),
        scratch_types=scratch,
    )
    def prop(table, srci, dsti, ewbi, zeros, out, sidx, didx, ewr, rows, acc,
             sem, *rest):
        cc = lax.axis_index("c")
        ss = lax.axis_index("s")
        wid = ss * 2 + cc
        pltpu.sync_copy(zeros.at[pl.ds(ss * NPT, NPT)], acc.at[pl.ds(ss * NPT, NPT)])
        if stage_table:
            stab = rest[0]

            @pl.when(ss < 10)
            def _():
                pltpu.sync_copy(table.at[pl.ds(ss * 1000, 1000)],
                                stab.at[pl.ds(ss * 1000, 1000)])
            gsrc = stab
        else:
            gsrc = table
        plsc.subcore_barrier()

        ebase = wid * EPW

        def chunk_body(i, carry):
            base = ebase + i * chunk
            pltpu.sync_copy(srci.at[pl.ds(base, chunk)], sidx)
            pltpu.sync_copy(dsti.at[pl.ds(base, chunk)], didx)
            pltpu.sync_copy(ewbi.at[pl.ds(base, chunk)], ewr)
            pltpu.async_copy(gsrc.at[sidx], rows, sem).wait()

            for e in range(chunk):
                cb = ewr[e, pl.ds(0, 16)]
                for f in range(F // 16):
                    rows[e, pl.ds(f * 16, 16)] = rows[e, pl.ds(f * 16, 16)] * cb
            pltpu.sync_copy(rows, acc.at[didx], add=True)
            return carry

        lax.fori_loop(0, EPW // chunk, chunk_body, 0)
        plsc.subcore_barrier()
        pltpu.sync_copy(acc.at[pl.ds(ss * NPT, NPT)],
                        out.at[cc, pl.ds(ss * NPT, NPT)])

    return prop


_RB = 1000
_GRID = N // _RB

_EB = 10000


def _tc_call(body, in_arrays, in_specs, out_shapes, out_specs):
    return pl.pallas_call(
        body,
        grid=(_GRID,),
        in_specs=in_specs,
        out_specs=out_specs,
        out_shape=out_shapes,
    )(*in_arrays)


def _rows_spec(f):
    return pl.BlockSpec((_RB, f), lambda i: (i, 0))


def _parts_spec(f):
    return pl.BlockSpec((2, _RB, f), lambda i: (0, i, 0))


def _full_spec(a, b):
    return pl.BlockSpec((a, b), lambda i: (0, 0))


def _ewb_body(w, o):
    o[...] = jnp.broadcast_to(w[...], (w.shape[0], 16))


def _prep_body(d0, x, dinv_o, xs_o):
    d = d0[...]
    dinv = jnp.where(d > 0, 1.0 / jnp.sqrt(jnp.maximum(d, 1e-12)), 0.0)
    dinv_o[...] = dinv
    xs_o[...] = x[...] * dinv


def _combine_body(parts, dinv, tx1_o, u_o):
    t = -(dinv[...] * (parts[0] + parts[1]))
    tx1_o[...] = t
    u_o[...] = dinv[...] * t


def _gates1_body(x, tx1, parts2, dinv, w1, b1c, wc1r, c1w, c1b, h_o, hs_o):
    tp = -(dinv[...] * (parts2[0] + parts2[1]))
    tx2 = 2.0 * tp - x[...]
    a = jnp.concatenate([x[...], tx1[...], tx2], axis=1)
    g = jnp.dot(a, w1[...], preferred_element_type=jnp.float32) + b1c[...]
    g0, g2, g3 = g[:, :64], g[:, 64:128], g[:, 128:]
    cn = jax.nn.sigmoid(g0) * jnp.tanh(g2)
    o = jax.nn.sigmoid(g3 + wc1r[...] * cn)
    h = jax.nn.relu(o * jnp.tanh(cn))
    h1 = jax.nn.relu(
        jnp.dot(h, c1w[...], preferred_element_type=jnp.float32) + c1b[...])
    h_o[...] = h1
    hs_o[...] = jnp.concatenate(
        [h1 * dinv[...], jnp.zeros((h1.shape[0], 96), h1.dtype)], axis=1)


def _final_body(h1, th1, parts4, dinv, w2, b2c, wc2r, c2w, c2b, linw, linb, o_out):
    tp = -(dinv[...] * (parts4[0, :, :32] + parts4[1, :, :32]))
    th2 = 2.0 * tp - h1[...]
    a = jnp.concatenate([h1[...], th1[..., :32], th2], axis=1)
    g = jnp.dot(a, w2[...], preferred_element_type=jnp.float32) + b2c[...]
    g0, g2, g3 = g[:, :16], g[:, 16:32], g[:, 32:]
    cn = jax.nn.sigmoid(g0) * jnp.tanh(g2)
    o = jax.nn.sigmoid(g3 + wc2r[...] * cn)
    h = jax.nn.relu(o * jnp.tanh(cn))
    o128 = jax.nn.relu(
        jnp.dot(h, c2w[...], preferred_element_type=jnp.float32) + c2b[...])
    o_out[...] = (
        jnp.dot(o128, linw[...], preferred_element_type=jnp.float32) + linb[...])


def kernel(x, edge_index, edge_weight, wx1, wh1, wc1, b1, wx2, wh2, wc2, b2,
           conv1_w, conv1_b, conv2_w, conv2_b, lin_w, lin_b):
    src = edge_index[0].astype(jnp.int32)
    dst = edge_index[1].astype(jnp.int32)
    ew = edge_weight.astype(jnp.float32)

    zeros128 = jnp.zeros((NP, 128), jnp.float32)

    w1 = jnp.concatenate(
        [jnp.concatenate([wx1[g, k] for g in (0, 2, 3)], axis=1)
         for k in range(3)], axis=0)
    b1c = jnp.concatenate([b1[0], b1[2], b1[3]])[None, :]
    w2 = jnp.concatenate(
        [jnp.concatenate([wx2[g, k] for g in (0, 2, 3)], axis=1)
         for k in range(3)], axis=0)
    b2c = jnp.concatenate([b2[0], b2[2], b2[3]])[None, :]
    wc1r = wc1[2][None, :]
    wc2r = wc2[2][None, :]
    c1w = conv1_w.T
    c1b = conv1_b[None, :]
    c2w = conv2_w.T
    c2b = conv2_b[None, :]
    linw = lin_w.T
    linb = lin_b[None, :]

    ewb = pl.pallas_call(
        _ewb_body,
        grid=(E // _EB,),
        in_specs=[pl.BlockSpec((_EB, 1), lambda i: (i, 0))],
        out_specs=pl.BlockSpec((_EB, 16), lambda i: (i, 0)),
        out_shape=jax.ShapeDtypeStruct((E, 16), jnp.float32),
    )(ew[:, None])

    prop128 = _make_propagate(128, 40, stage_table=False)

    ones128 = jnp.ones((N, 128), jnp.float32)
    degp = prop128(ones128, src, dst, ewb, zeros128)
    degsum = (degp[0, :, 0] + degp[1, :, 0])[:, None]

    dinv, xs = _tc_call(
        _prep_body,
        [degsum, x],
        [_rows_spec(1), _rows_spec(128)],
        [jax.ShapeDtypeStruct((N, 1), jnp.float32),
         jax.ShapeDtypeStruct((N, 128), jnp.float32)],
        [_rows_spec(1), _rows_spec(128)],
    )

    p1 = prop128(xs, src, dst, ewb, zeros128)
    tx1, u2 = _tc_call(
        _combine_body,
        [p1, dinv],
        [_parts_spec(128), _rows_spec(1)],
        [jax.ShapeDtypeStruct((N, 128), jnp.float32),
         jax.ShapeDtypeStruct((N, 128), jnp.float32)],
        [_rows_spec(128), _rows_spec(128)],
    )
    p2 = prop128(u2, src, dst, ewb, zeros128)
    h1, hs = _tc_call(
        _gates1_body,
        [x, tx1, p2, dinv, w1, b1c, wc1r, c1w, c1b],
        [_rows_spec(128), _rows_spec(128), _parts_spec(128), _rows_spec(1),
         _full_spec(384, 192), _full_spec(1, 192), _full_spec(1, 64),
         _full_spec(64, 32), _full_spec(1, 32)],
        [jax.ShapeDtypeStruct((N, 32), jnp.float32),
         jax.ShapeDtypeStruct((N, 128), jnp.float32)],
        [_rows_spec(32), _rows_spec(128)],
    )

    p3 = prop128(hs, src, dst, ewb, zeros128)
    th1, u4 = _tc_call(
        _combine_body,
        [p3, dinv],
        [_parts_spec(128), _rows_spec(1)],
        [jax.ShapeDtypeStruct((N, 128), jnp.float32),
         jax.ShapeDtypeStruct((N, 128), jnp.float32)],
        [_rows_spec(128), _rows_spec(128)],
    )
    p4 = prop128(u4, src, dst, ewb, zeros128)
    o = _tc_call(
        _final_body,
        [h1, th1, p4, dinv, w2, b2c, wc2r, c2w, c2b, linw, linb],
        [_rows_spec(32), _rows_spec(128), _parts_spec(128), _rows_spec(1),
         _full_spec(96, 48), _full_spec(1, 48), _full_spec(1, 16),
         _full_spec(16, 128), _full_spec(1, 128), _full_spec(128, 1),
         _full_spec(1, 1)],
        [jax.ShapeDtypeStruct((N, 1), jnp.float32)],
        [_rows_spec(1)],
    )[0]

    return jnp.concatenate([o, o], axis=0)


# speedup vs baseline: 3.1499x; 1.0957x over previous
import functools

import jax
import jax.numpy as jnp
from jax import lax
from jax.experimental import pallas as pl
from jax.experimental.pallas import tpu as pltpu
from jax.experimental.pallas import tpu_sc as plsc

N = 10000
NP = 10112
E = 320000
NTILES = 32
NSUB = 16
EPW = E // NTILES
CHUNK = 80
NPT = NP // NSUB


def _sc_mesh():
    return plsc.VectorSubcoreMesh(core_axis_name="c", subcore_axis_name="s")


def _make_propagate(F, chunk, stage_table):
    scratch = [
        pltpu.VMEM((chunk,), jnp.int32),
        pltpu.VMEM((chunk,), jnp.int32),
        pltpu.VMEM((chunk, 16), jnp.float32),
        pltpu.VMEM((chunk, F), jnp.float32),
        pltpu.VMEM_SHARED((NP, F), jnp.float32),
        pltpu.SemaphoreType.DMA,
    ]
    if stage_table:
        scratch.append(pltpu.VMEM_SHARED((NP, F), jnp.float32))

    @functools.partial(
        pl.kernel,
        out_type=jax.ShapeDtypeStruct((2, NP, F), jnp.float32),
        mesh=_sc_mesh(),
        scratch_types=scratch,
    )
    def prop(table, srci, dsti, ewbi, zeros, out, sidx, didx, ewr, rows, acc,
             sem, *rest):
        cc = lax.axis_index("c")
        ss = lax.axis_index("s")
        wid = ss * 2 + cc
        pltpu.sync_copy(zeros.at[pl.ds(ss * NPT, NPT)], acc.at[pl.ds(ss * NPT, NPT)])
        if stage_table:
            stab = rest[0]

            @pl.when(ss < 10)
            def _():
                pltpu.sync_copy(table.at[pl.ds(ss * 1000, 1000)],
                                stab.at[pl.ds(ss * 1000, 1000)])
            gsrc = stab
        else:
            gsrc = table
        plsc.subcore_barrier()

        ebase = wid * EPW

        def chunk_body(i, carry):
            base = ebase + i * chunk
            pltpu.sync_copy(srci.at[pl.ds(base, chunk)], sidx)
            pltpu.sync_copy(dsti.at[pl.ds(base, chunk)], didx)
            pltpu.sync_copy(ewbi.at[pl.ds(base, chunk)], ewr)
            pltpu.async_copy(gsrc.at[sidx], rows, sem).wait()

            for e in range(chunk):
                cb = ewr[e, pl.ds(0, 16)]
                for f in range(F // 16):
                    rows[e, pl.ds(f * 16, 16)] = rows[e, pl.ds(f * 16, 16)] * cb
            pltpu.sync_copy(rows, acc.at[didx], add=True)
            return carry

        lax.fori_loop(0, EPW // chunk, chunk_body, 0)
        plsc.subcore_barrier()
        pltpu.sync_copy(acc.at[pl.ds(ss * NPT, NPT)],
                        out.at[cc, pl.ds(ss * NPT, NPT)])

    return prop


def _make_degree(chunk):
    @functools.partial(
        pl.kernel,
        out_type=jax.ShapeDtypeStruct((2, NP, 128), jnp.float32),
        mesh=_sc_mesh(),
        scratch_types=[
            pltpu.VMEM((chunk,), jnp.int32),
            pltpu.VMEM((chunk, 16), jnp.float32),
            pltpu.VMEM((chunk, 128), jnp.float32),
            pltpu.VMEM_SHARED((NP, 128), jnp.float32),
        ],
    )
    def deg(dsti, ewbi, zeros, out, didx, ewr, rows, acc):
        cc = lax.axis_index("c")
        ss = lax.axis_index("s")
        wid = ss * 2 + cc
        pltpu.sync_copy(zeros.at[pl.ds(ss * NPT, NPT)], acc.at[pl.ds(ss * NPT, NPT)])
        pltpu.sync_copy(zeros.at[pl.ds(0, chunk)], rows)
        plsc.subcore_barrier()

        ebase = wid * EPW

        def chunk_body(i, carry):
            base = ebase + i * chunk
            pltpu.sync_copy(dsti.at[pl.ds(base, chunk)], didx)
            pltpu.sync_copy(ewbi.at[pl.ds(base, chunk)], ewr)
            for e in range(chunk):
                rows[e, pl.ds(0, 16)] = ewr[e, pl.ds(0, 16)]
            pltpu.sync_copy(rows, acc.at[didx], add=True)
            return carry

        lax.fori_loop(0, EPW // chunk, chunk_body, 0)
        plsc.subcore_barrier()
        pltpu.sync_copy(acc.at[pl.ds(ss * NPT, NPT)],
                        out.at[cc, pl.ds(ss * NPT, NPT)])

    return deg


_RB = 1000
_GRID = N // _RB

_EB = 10000


def _tc_call(body, in_arrays, in_specs, out_shapes, out_specs):
    return pl.pallas_call(
        body,
        grid=(_GRID,),
        in_specs=in_specs,
        out_specs=out_specs,
        out_shape=out_shapes,
    )(*in_arrays)


def _rows_spec(f):
    return pl.BlockSpec((_RB, f), lambda i: (i, 0))


def _parts_spec(f):
    return pl.BlockSpec((2, _RB, f), lambda i: (0, i, 0))


def _full_spec(a, b):
    return pl.BlockSpec((a, b), lambda i: (0, 0))


def _ewb_body(w, o):
    o[...] = jnp.broadcast_to(w[...], (w.shape[0], 16))


def _prep_body(d0, x, dinv_o, xs_o):
    d = d0[...]
    dinv = jnp.where(d > 0, 1.0 / jnp.sqrt(jnp.maximum(d, 1e-12)), 0.0)
    dinv_o[...] = dinv
    xs_o[...] = x[...] * dinv


def _combine_body(parts, dinv, tx1_o, u_o):
    t = -(dinv[...] * (parts[0] + parts[1]))
    tx1_o[...] = t
    u_o[...] = dinv[...] * t


def _gates1_body(x, tx1, parts2, dinv, w1, b1c, wc1r, c1w, c1b, h_o, hs_o):
    tp = -(dinv[...] * (parts2[0] + parts2[1]))
    tx2 = 2.0 * tp - x[...]
    a = jnp.concatenate([x[...], tx1[...], tx2], axis=1)
    g = jnp.dot(a, w1[...], preferred_element_type=jnp.float32) + b1c[...]
    g0, g2, g3 = g[:, :64], g[:, 64:128], g[:, 128:]
    cn = jax.nn.sigmoid(g0) * jnp.tanh(g2)
    o = jax.nn.sigmoid(g3 + wc1r[...] * cn)
    h = jax.nn.relu(o * jnp.tanh(cn))
    h1 = jax.nn.relu(
        jnp.dot(h, c1w[...], preferred_element_type=jnp.float32) + c1b[...])
    h_o[...] = h1
    hs_o[...] = jnp.concatenate(
        [h1 * dinv[...], jnp.zeros((h1.shape[0], 96), h1.dtype)], axis=1)


def _final_body(h1, th1, parts4, dinv, w2, b2c, wc2r, c2w, c2b, linw, linb, o_out):
    tp = -(dinv[...] * (parts4[0, :, :32] + parts4[1, :, :32]))
    th2 = 2.0 * tp - h1[...]
    a = jnp.concatenate([h1[...], th1[..., :32], th2], axis=1)
    g = jnp.dot(a, w2[...], preferred_element_type=jnp.float32) + b2c[...]
    g0, g2, g3 = g[:, :16], g[:, 16:32], g[:, 32:]
    cn = jax.nn.sigmoid(g0) * jnp.tanh(g2)
    o = jax.nn.sigmoid(g3 + wc2r[...] * cn)
    h = jax.nn.relu(o * jnp.tanh(cn))
    o128 = jax.nn.relu(
        jnp.dot(h, c2w[...], preferred_element_type=jnp.float32) + c2b[...])
    o_out[...] = (
        jnp.dot(o128, linw[...], preferred_element_type=jnp.float32) + linb[...])


def kernel(x, edge_index, edge_weight, wx1, wh1, wc1, b1, wx2, wh2, wc2, b2,
           conv1_w, conv1_b, conv2_w, conv2_b, lin_w, lin_b):
    src = edge_index[0].astype(jnp.int32)
    dst = edge_index[1].astype(jnp.int32)
    ew = edge_weight.astype(jnp.float32)

    zeros128 = jnp.zeros((NP, 128), jnp.float32)

    w1 = jnp.concatenate(
        [jnp.concatenate([wx1[g, k] for g in (0, 2, 3)], axis=1)
         for k in range(3)], axis=0)
    b1c = jnp.concatenate([b1[0], b1[2], b1[3]])[None, :]
    w2 = jnp.concatenate(
        [jnp.concatenate([wx2[g, k] for g in (0, 2, 3)], axis=1)
         for k in range(3)], axis=0)
    b2c = jnp.concatenate([b2[0], b2[2], b2[3]])[None, :]
    wc1r = wc1[2][None, :]
    wc2r = wc2[2][None, :]
    c1w = conv1_w.T
    c1b = conv1_b[None, :]
    c2w = conv2_w.T
    c2b = conv2_b[None, :]
    linw = lin_w.T
    linb = lin_b[None, :]

    ewb = pl.pallas_call(
        _ewb_body,
        grid=(E // _EB,),
        in_specs=[pl.BlockSpec((_EB, 1), lambda i: (i, 0))],
        out_specs=pl.BlockSpec((_EB, 16), lambda i: (i, 0)),
        out_shape=jax.ShapeDtypeStruct((E, 16), jnp.float32),
    )(ew[:, None])

    prop128 = _make_propagate(128, 40, stage_table=False)

    degp = _make_degree(40)(dst, ewb, zeros128)
    degsum = (degp[0, :, 0] + degp[1, :, 0])[:, None]

    dinv, xs = _tc_call(
        _prep_body,
        [degsum, x],
        [_rows_spec(1), _rows_spec(128)],
        [jax.ShapeDtypeStruct((N, 1), jnp.float32),
         jax.ShapeDtypeStruct((N, 128), jnp.float32)],
        [_rows_spec(1), _rows_spec(128)],
    )

    p1 = prop128(xs, src, dst, ewb, zeros128)
    tx1, u2 = _tc_call(
        _combine_body,
        [p1, dinv],
        [_parts_spec(128), _rows_spec(1)],
        [jax.ShapeDtypeStruct((N, 128), jnp.float32),
         jax.ShapeDtypeStruct((N, 128), jnp.float32)],
        [_rows_spec(128), _rows_spec(128)],
    )
    p2 = prop128(u2, src, dst, ewb, zeros128)
    h1, hs = _tc_call(
        _gates1_body,
        [x, tx1, p2, dinv, w1, b1c, wc1r, c1w, c1b],
        [_rows_spec(128), _rows_spec(128), _parts_spec(128), _rows_spec(1),
         _full_spec(384, 192), _full_spec(1, 192), _full_spec(1, 64),
         _full_spec(64, 32), _full_spec(1, 32)],
        [jax.ShapeDtypeStruct((N, 32), jnp.float32),
         jax.ShapeDtypeStruct((N, 128), jnp.float32)],
        [_rows_spec(32), _rows_spec(128)],
    )

    p3 = prop128(hs, src, dst, ewb, zeros128)
    th1, u4 = _tc_call(
        _combine_body,
        [p3, dinv],
        [_parts_spec(128), _rows_spec(1)],
        [jax.ShapeDtypeStruct((N, 128), jnp.float32),
         jax.ShapeDtypeStruct((N, 128), jnp.float32)],
        [_rows_spec(128), _rows_spec(128)],
    )
    p4 = prop128(u4, src, dst, ewb, zeros128)
    o = _tc_call(
        _final_body,
        [h1, th1, p4, dinv, w2, b2c, wc2r, c2w, c2b, linw, linb],
        [_rows_spec(32), _rows_spec(128), _parts_spec(128), _rows_spec(1),
         _full_spec(96, 48), _full_spec(1, 48), _full_spec(1, 16),
         _full_spec(16, 128), _full_spec(1, 128), _full_spec(128, 1),
         _full_spec(1, 1)],
        [jax.ShapeDtypeStruct((N, 1), jnp.float32)],
        [_rows_spec(1)],
    )[0]

    return jnp.concatenate([o, o], axis=0)
